# Initial kernel scaffold; baseline (speedup 1.0000x reference)
import jax
import jax.numpy as jnp
from jax.experimental import pallas as pl


def kernel(positions, species, charges, atom_mask, W_in, W_rad, W_prev, W_self, W_msg, W_top1, b_top1, W_top2, b_top2):
    raise NotImplementedError("write your pallas kernel here")


# speedup vs baseline: 2.6682x; 2.6682x over previous
import functools

import jax
import jax.numpy as jnp
from jax.experimental import pallas as pl
from jax.experimental.pallas import tpu as pltpu

_NUM_CG = 3
_TAU = 16
_NUM_SPECIES = 5
_CHARGE_POWER = 2
_NUM_BASIS = 10
_CHARGE_SCALE = 9.0
_SOFT_CUT_RAD = 1.73
_SOFT_CUT_WIDTH = 0.2
_HARD_CUT_RAD = 100.0
_N = 64
_NSI = _NUM_SPECIES * (_CHARGE_POWER + 1)
_KPAD = 16

def _edge_kernel(pos_ref, spec_ref, chg_ref, am_ref,
                 win_ref, wrad_ref, wprev_ref, wself_ref, wmsg_ref,
                 w1_ref, b1_ref, w2_ref, b2_ref, out_ref):
    N, T = _N, _TAU
    R = N * N
    f32 = jnp.float32

    pos = pos_ref[0]
    am = am_ref[0]

    rel = pos[:, None, :] - pos[None, :, :]
    dist2 = jnp.sum(rel * rel, axis=-1, keepdims=True)
    norms = jnp.sqrt(jnp.maximum(dist2, 1e-12))

    ii = jax.lax.broadcasted_iota(jnp.int32, (N, N, 1), 0)
    jj = jax.lax.broadcasted_iota(jnp.int32, (N, N, 1), 1)
    off_diag = (ii != jj).astype(f32)
    emask = am[:, None, :] * am[None, :, :] * off_diag

    cut = (jax.nn.sigmoid((_SOFT_CUT_RAD - norms) / _SOFT_CUT_WIDTH)
           * emask * (norms < _HARD_CUT_RAD).astype(f32))

    ctr = jax.lax.broadcasted_iota(
        jnp.int32, (N, N, _KPAD), 2).astype(f32) * (4.0 / 9.0)
    dctr = norms - ctr
    basis = jnp.exp(dctr * dctr * (-1.0 / (2.0 * 0.3 * 0.3)))

    basis_f = basis.reshape(R, _KPAD)
    cut_f = cut.reshape(R, 1)
    emask_f = emask.reshape(R, 1)

    sp = spec_ref[0]
    c = chg_ref[0] * (1.0 / _CHARGE_SCALE)
    lane = jax.lax.broadcasted_iota(jnp.int32, (N, _KPAD), 1)
    onehot = ((sp == lane // (_CHARGE_POWER + 1))
              & (lane < _NSI)).astype(f32)
    p = lane % (_CHARGE_POWER + 1)
    cb = jnp.broadcast_to(c, (N, _KPAD))
    cpow = jnp.where(p == 0, 1.0, jnp.where(p == 1, cb, cb * cb))
    scal = onehot * cpow * am

    a = jnp.dot(scal, win_ref[...], preferred_element_type=f32)

    e_levels = []
    e_prev = None
    for lvl in range(_NUM_CG):
        rad = jnp.dot(basis_f, wrad_ref[lvl],
                      preferred_element_type=f32)
        dots = (a[:, None, :] * a[None, :, :]).reshape(R, T)
        if e_prev is None:
            pre = dots
        else:
            pre = dots + jnp.dot(e_prev, wprev_ref[lvl],
                                 preferred_element_type=f32)
        e = pre * rad * cut_f
        msg = jnp.sum(e.reshape(N, N, T), axis=1)
        a = (jnp.dot(a, wself_ref[lvl], preferred_element_type=f32)
             + jnp.dot(msg, wmsg_ref[lvl], preferred_element_type=f32)) * am
        e_levels.append(e)
        e_prev = e

    feat = jnp.concatenate(e_levels, axis=1)
    h = jnp.dot(feat, w1_ref[...], preferred_element_type=f32) + b1_ref[0:1, :]
    h = jnp.where(h >= 0.0, h, 0.01 * h)
    pred = jnp.dot(h, w2_ref[...], preferred_element_type=f32) + b2_ref[0:1, 0:1]
    pred = pred * emask_f
    out_ref[...] = pred.reshape(1, N, N, 1)


@functools.partial(jax.jit, static_argnames=())
def kernel(positions, species, charges, atom_mask,
           W_in, W_rad, W_prev, W_self, W_msg,
           W_top1, b_top1, W_top2, b_top2):
    B, N = positions.shape[0], positions.shape[1]
    T = _TAU
    f32 = jnp.float32

    spec3 = species.astype(jnp.int32).reshape(B, N, 1)
    chg3 = charges.astype(f32).reshape(B, N, 1)
    am3 = atom_mask.astype(f32).reshape(B, N, 1)

    W_in_p = jnp.zeros((_KPAD, T), f32).at[:_NSI].set(W_in.astype(f32))
    W_rad_p = jnp.zeros((_NUM_CG, _KPAD, T), f32).at[:, :_NUM_BASIS].set(
        W_rad.astype(f32))
    W_self0 = W_self[:, 0].astype(f32)
    W_msg0 = W_msg[:, 0].astype(f32)
    sl = 3 * T
    W1_eff = jnp.concatenate(
        [W_top1[lvl * sl: lvl * sl + T] for lvl in range(_NUM_CG)],
        axis=0).astype(f32)
    HID = W1_eff.shape[1]
    b1 = jnp.broadcast_to(b_top1.astype(f32)[None, :], (8, HID))
    b2 = jnp.broadcast_to(b_top2.astype(f32)[None, :], (8, 128))
    W2 = W_top2.astype(f32)

    full = lambda shape: pl.BlockSpec(shape, lambda b: (0,) * len(shape))

    out = pl.pallas_call(
        _edge_kernel,
        grid=(B,),
        in_specs=[
            pl.BlockSpec((1, N, 3), lambda b: (b, 0, 0)),
            pl.BlockSpec((1, N, 1), lambda b: (b, 0, 0)),
            pl.BlockSpec((1, N, 1), lambda b: (b, 0, 0)),
            pl.BlockSpec((1, N, 1), lambda b: (b, 0, 0)),
            full((_KPAD, T)),
            full((_NUM_CG, _KPAD, T)),
            full((_NUM_CG, T, T)),
            full((_NUM_CG, T, T)),
            full((_NUM_CG, T, T)),
            full((3 * T, HID)),
            full((8, HID)),
            full((HID, 1)),
            full((8, 128)),
        ],
        out_specs=pl.BlockSpec((1, N, N, 1), lambda b: (b, 0, 0, 0)),
        out_shape=jax.ShapeDtypeStruct((B, N, N, 1), f32),
        compiler_params=pltpu.CompilerParams(
            dimension_semantics=("parallel",)),
    )(positions.astype(f32), spec3, chg3, am3,
      W_in_p, W_rad_p, W_prev.astype(f32), W_self0, W_msg0,
      W1_eff, b1, W2, b2)
    return out


# speedup vs baseline: 7.6451x; 2.8653x over previous
import jax
import jax.numpy as jnp
from jax.experimental import pallas as pl
from jax.experimental.pallas import tpu as pltpu

_NUM_CG = 3
_TAU = 16
_NUM_SPECIES = 5
_CHARGE_POWER = 2
_NUM_BASIS = 10
_CHARGE_SCALE = 9.0
_SOFT_CUT_RAD = 1.73
_SOFT_CUT_WIDTH = 0.2
_HARD_CUT_RAD = 100.0
_N = 64
_NSI = _NUM_SPECIES * (_CHARGE_POWER + 1)
_KPAD = 16
_G = 8
_L = _G * _KPAD


def _edge_kernel(pxyz_ref, spec_ref, chg_ref, amg_ref, am8_ref,
                 win_ref, wrad_ref, wprev_ref, wself_ref, wmsg_ref,
                 w1_ref, b1_ref, w2_ref, b2_ref, out_ref):
    N, L, G = _N, _L, _G
    R = N * N
    f32 = jnp.float32

    px = pxyz_ref[0, 0]
    py = pxyz_ref[0, 1]
    pz = pxyz_ref[0, 2]
    amg = amg_ref[0]

    dx = px[:, None, :] - px[None, :, :]
    dy = py[:, None, :] - py[None, :, :]
    dz = pz[:, None, :] - pz[None, :, :]
    dist2 = dx * dx + dy * dy + dz * dz
    norms = jnp.sqrt(jnp.maximum(dist2, 1e-12))

    ii = jax.lax.broadcasted_iota(jnp.int32, (N, N, 1), 0)
    jj = jax.lax.broadcasted_iota(jnp.int32, (N, N, 1), 1)
    off_diag = (ii != jj).astype(f32)
    emask = amg[:, None, :] * amg[None, :, :] * off_diag

    cut = (jax.nn.sigmoid((_SOFT_CUT_RAD - norms) / _SOFT_CUT_WIDTH)
           * emask * (norms < _HARD_CUT_RAD).astype(f32))

    lane3 = jax.lax.broadcasted_iota(jnp.int32, (N, N, L), 2)
    ctr = (lane3 % _KPAD).astype(f32) * (4.0 / 9.0)
    dctr = norms - ctr
    basis = jnp.exp(dctr * dctr * (-1.0 / (2.0 * 0.3 * 0.3)))

    basis_f = basis.reshape(R, L)
    cut_f = cut.reshape(R, L)

    sp = spec_ref[0]
    c = chg_ref[0] * (1.0 / _CHARGE_SCALE)
    lane2 = jax.lax.broadcasted_iota(jnp.int32, (N, L), 1) % _KPAD
    onehot = ((sp == lane2 // (_CHARGE_POWER + 1))
              & (lane2 < _NSI)).astype(f32)
    p = lane2 % (_CHARGE_POWER + 1)
    cpow = jnp.where(p == 0, 1.0, jnp.where(p == 1, c, c * c))
    scal = onehot * cpow * amg

    a = jnp.dot(scal, win_ref[...], preferred_element_type=f32)

    h = None
    e_prev = None
    for lvl in range(_NUM_CG):
        rad = jnp.dot(basis_f, wrad_ref[lvl],
                      preferred_element_type=f32)
        dots = (a[:, None, :] * a[None, :, :]).reshape(R, L)
        if e_prev is None:
            pre = dots
        else:
            pre = dots + jnp.dot(e_prev, wprev_ref[lvl],
                                 preferred_element_type=f32)
        e = pre * rad * cut_f
        msg = jnp.sum(e.reshape(N, N, L), axis=1)
        a = (jnp.dot(a, wself_ref[lvl], preferred_element_type=f32)
             + jnp.dot(msg, wmsg_ref[lvl], preferred_element_type=f32)) * amg
        hc = jnp.dot(e, w1_ref[lvl], preferred_element_type=f32)
        h = hc if h is None else h + hc
        e_prev = e

    h = h + b1_ref[0:1, :]
    h = jnp.where(h >= 0.0, h, 0.01 * h)
    pred = (jnp.dot(h, w2_ref[...], preferred_element_type=f32)
            + b2_ref[0:1, 0:1])

    am8 = am8_ref[0]
    em8 = (am8[:, None, :] * am8[None, :, :] * off_diag).reshape(R, G)
    out_ref[...] = (pred * em8).reshape(1, N, N, G)


def kernel(positions, species, charges, atom_mask,
           W_in, W_rad, W_prev, W_self, W_msg,
           W_top1, b_top1, W_top2, b_top2):
    B, N = positions.shape[0], positions.shape[1]
    T, G, L = _TAU, _G, _L
    NB = B // G
    f32 = jnp.float32
    eye8 = jnp.eye(G, dtype=f32)

    def pack(x):
        return jnp.repeat(
            x.reshape(NB, G, N).transpose(0, 2, 1), _KPAD, axis=-1)

    pxyz = jnp.stack([pack(positions[..., k].astype(f32)) for k in range(3)],
                     axis=1)
    spc = pack(species.astype(jnp.int32))
    chg = pack(charges.astype(f32))
    amf = atom_mask.astype(f32)
    amg = pack(amf)
    am8 = amf.reshape(NB, G, N).transpose(0, 2, 1)

    def blk(W):
        return jnp.kron(eye8, W.astype(f32))

    W_in_p = jnp.zeros((_KPAD, T), f32).at[:_NSI].set(W_in.astype(f32))
    W_rad_p = jnp.zeros((_NUM_CG, _KPAD, T), f32).at[:, :_NUM_BASIS].set(
        W_rad.astype(f32))
    Win_b = blk(W_in_p)
    Wrad_b = jnp.stack([blk(W_rad_p[i]) for i in range(_NUM_CG)])
    Wprev_b = jnp.stack([blk(W_prev[i]) for i in range(_NUM_CG)])
    Wself_b = jnp.stack([blk(W_self[i, 0]) for i in range(_NUM_CG)])
    Wmsg_b = jnp.stack([blk(W_msg[i, 0]) for i in range(_NUM_CG)])

    sl = (2 + 1) * T
    HID = W_top1.shape[1]
    W1_3 = jnp.stack([W_top1[lvl * sl: lvl * sl + T] for lvl in range(_NUM_CG)]
                     ).astype(f32)
    W1_g = jnp.einsum('ltu,bc->lbtcu', W1_3, eye8).reshape(
        _NUM_CG, L, G * HID)
    b1_g = jnp.broadcast_to(jnp.tile(b_top1.astype(f32), G)[None, :],
                            (8, G * HID))
    W2_g = jnp.einsum('u,bc->buc', W_top2[:, 0].astype(f32), eye8).reshape(
        G * HID, G)
    b2_g = jnp.broadcast_to(b_top2.astype(f32).reshape(1, 1), (8, 128))

    full = lambda shape: pl.BlockSpec(shape, lambda b: (0,) * len(shape))

    out = pl.pallas_call(
        _edge_kernel,
        grid=(NB,),
        in_specs=[
            pl.BlockSpec((1, 3, N, L), lambda b: (b, 0, 0, 0)),
            pl.BlockSpec((1, N, L), lambda b: (b, 0, 0)),
            pl.BlockSpec((1, N, L), lambda b: (b, 0, 0)),
            pl.BlockSpec((1, N, L), lambda b: (b, 0, 0)),
            pl.BlockSpec((1, N, G), lambda b: (b, 0, 0)),
            full((L, L)),
            full((_NUM_CG, L, L)),
            full((_NUM_CG, L, L)),
            full((_NUM_CG, L, L)),
            full((_NUM_CG, L, L)),
            full((_NUM_CG, L, G * HID)),
            full((8, G * HID)),
            full((G * HID, G)),
            full((8, 128)),
        ],
        out_specs=pl.BlockSpec((1, N, N, G), lambda b: (b, 0, 0, 0)),
        out_shape=jax.ShapeDtypeStruct((NB, N, N, G), f32),
        compiler_params=pltpu.CompilerParams(
            dimension_semantics=("parallel",)),
    )(pxyz, spc, chg, amg, am8,
      Win_b, Wrad_b, Wprev_b, Wself_b, Wmsg_b, W1_g, b1_g, W2_g, b2_g)

    return out.transpose(0, 3, 1, 2).reshape(B, N, N, 1)
